# baseline (speedup 1.0000x reference)
import functools

import jax
import jax.numpy as jnp
from jax import lax
from jax.experimental import pallas as pl
from jax.experimental.pallas import tpu as pltpu
from jax.experimental.pallas import tpu_sc as plsc

NUM_K = 8192
DIM = 256
NUM_N = 16384
TN = 256
STEPS = NUM_N // TN

SC_CORES = 2
SC_SUBCORES = 16
SC_WORKERS = SC_CORES * SC_SUBCORES
ROWS_PER_W = NUM_N // SC_WORKERS
GCHUNK = 128


def _dist_body(z_ref, et_ref, idx_ref, loss_ref, perp_ref,
               esq_ref, counts_ref, acc_ref):
    i = pl.program_id(0)
    et = et_ref[...]

    @pl.when(i == 0)
    def _init():
        esq_ref[...] = jnp.sum(et * et, axis=0, keepdims=True)
        counts_ref[...] = jnp.zeros((1, NUM_K), jnp.float32)
        acc_ref[0] = 0.0

    z = z_ref[...]
    z_sq = jnp.sum(z * z, axis=1, keepdims=True)
    s = lax.dot_general(z, et, (((1,), (0,)), ((), ())),
                        preferred_element_type=jnp.float32)
    d2 = (z_sq + esq_ref[...]) - 2.0 * s
    dist = jnp.sqrt(jnp.clip(d2, 0.0, None))
    minv = jnp.min(dist, axis=1, keepdims=True)
    kiota = lax.broadcasted_iota(jnp.int32, (TN, NUM_K), 1)
    idxs = jnp.min(jnp.where(dist == minv, kiota, NUM_K),
                   axis=1, keepdims=True)
    idx_ref[...] = idxs

    acc_ref[0] += jnp.sum(minv * minv)
    counts_ref[...] += jnp.sum((idxs == kiota).astype(jnp.float32),
                               axis=0, keepdims=True)

    @pl.when(i == STEPS - 1)
    def _fini():
        ones11 = jnp.ones((1, 1), jnp.float32)
        loss_ref[...] = (0.1 * (acc_ref[0] / (NUM_N * DIM))) * ones11
        p = counts_ref[...] / float(NUM_N)
        ent = jnp.sum(p * jnp.log(p + 1e-10), axis=1, keepdims=True)
        perp_ref[...] = jnp.exp(-ent)


def _dist_call(z_e, et):
    return pl.pallas_call(
        _dist_body,
        grid=(STEPS,),
        in_specs=[
            pl.BlockSpec((TN, DIM), lambda i: (i, 0)),
            pl.BlockSpec((DIM, NUM_K), lambda i: (0, 0)),
        ],
        out_specs=[
            pl.BlockSpec((TN, 1), lambda i: (i, 0)),
            pl.BlockSpec((1, 1), lambda i: (0, 0)),
            pl.BlockSpec((1, 1), lambda i: (0, 0)),
        ],
        out_shape=[
            jax.ShapeDtypeStruct((NUM_N, 1), jnp.int32),
            jax.ShapeDtypeStruct((1, 1), jnp.float32),
            jax.ShapeDtypeStruct((1, 1), jnp.float32),
        ],
        scratch_shapes=[
            pltpu.VMEM((1, NUM_K), jnp.float32),
            pltpu.VMEM((1, NUM_K), jnp.float32),
            pltpu.SMEM((1,), jnp.float32),
        ],
        compiler_params=pltpu.CompilerParams(
            dimension_semantics=("arbitrary",)),
    )(z_e, et)


def _gather_body(e_hbm, idx_hbm, out_hbm, idx_c, rows_v, sem):
    c = lax.axis_index("c")
    s = lax.axis_index("s")
    wid = s * SC_CORES + c
    base = wid * ROWS_PER_W
    for ch in range(ROWS_PER_W // GCHUNK):
        off = base + ch * GCHUNK
        pltpu.sync_copy(idx_hbm.at[pl.ds(off, GCHUNK)], idx_c)
        pltpu.async_copy(e_hbm.at[idx_c], rows_v, sem).wait()
        pltpu.sync_copy(rows_v, out_hbm.at[pl.ds(off, GCHUNK)])


def _gather_call(embeddings, indices):
    mesh = plsc.VectorSubcoreMesh(core_axis_name="c", subcore_axis_name="s")
    k = functools.partial(
        pl.kernel,
        mesh=mesh,
        out_type=jax.ShapeDtypeStruct((NUM_N, DIM), jnp.float32),
        scratch_types=[
            pltpu.VMEM((GCHUNK,), jnp.int32),
            pltpu.VMEM((GCHUNK, DIM), jnp.float32),
            pltpu.SemaphoreType.DMA,
        ],
    )(_gather_body)
    return k(embeddings, indices)


def kernel(z_e, embeddings):
    et = embeddings.T
    idx2, loss, perp = _dist_call(z_e, et)
    indices = idx2.reshape(NUM_N)
    z_q = _gather_call(embeddings, indices)
    return z_q, indices, loss[0, 0], perp[0, 0]


# speedup vs baseline: 1.0419x; 1.0419x over previous
import functools

import jax
import jax.numpy as jnp
from jax import lax
from jax.experimental import pallas as pl
from jax.experimental.pallas import tpu as pltpu
from jax.experimental.pallas import tpu_sc as plsc

NUM_K = 8192
DIM = 256
NUM_N = 16384
TN = 256
STEPS = NUM_N // TN

SC_CORES = 2
SC_SUBCORES = 16
SC_WORKERS = SC_CORES * SC_SUBCORES
ROWS_PER_W = NUM_N // SC_WORKERS
GCHUNK = 128


def _dist_body(z_ref, et2_ref, idx_ref, loss_ref, perp_ref,
               esq_ref, counts_ref, acc_ref):
    i = pl.program_id(0)
    et2 = et2_ref[...]

    @pl.when(i == 0)
    def _init():
        esq_ref[...] = 0.25 * jnp.sum(et2 * et2, axis=0, keepdims=True)
        counts_ref[...] = jnp.zeros((1, NUM_K), jnp.float32)
        acc_ref[0] = 0.0

    z = z_ref[...]
    z_sq = jnp.sum(z * z, axis=1, keepdims=True)
    s2 = lax.dot_general(z, et2, (((1,), (0,)), ((), ())),
                         preferred_element_type=jnp.float32)
    d2 = (z_sq + esq_ref[...]) - s2
    dist = jnp.sqrt(jnp.clip(d2, 0.0, None))
    minv = jnp.min(dist, axis=1, keepdims=True)
    kiota = lax.broadcasted_iota(jnp.int32, (TN, NUM_K), 1)
    idxs = jnp.min(jnp.where(dist == minv, kiota, NUM_K),
                   axis=1, keepdims=True)
    idx_ref[...] = idxs

    acc_ref[0] += jnp.sum(minv * minv)
    counts_ref[...] += jnp.sum((idxs == kiota).astype(jnp.float32),
                               axis=0, keepdims=True)

    @pl.when(i == STEPS - 1)
    def _fini():
        ones11 = jnp.ones((1, 1), jnp.float32)
        loss_ref[...] = (0.1 * (acc_ref[0] / (NUM_N * DIM))) * ones11
        p = counts_ref[...] / float(NUM_N)
        ent = jnp.sum(p * jnp.log(p + 1e-10), axis=1, keepdims=True)
        perp_ref[...] = jnp.exp(-ent)


def _dist_call(z_e, et):
    return pl.pallas_call(
        _dist_body,
        grid=(STEPS,),
        in_specs=[
            pl.BlockSpec((TN, DIM), lambda i: (i, 0)),
            pl.BlockSpec((DIM, NUM_K), lambda i: (0, 0)),
        ],
        out_specs=[
            pl.BlockSpec((TN, 1), lambda i: (i, 0)),
            pl.BlockSpec((1, 1), lambda i: (0, 0)),
            pl.BlockSpec((1, 1), lambda i: (0, 0)),
        ],
        out_shape=[
            jax.ShapeDtypeStruct((NUM_N, 1), jnp.int32),
            jax.ShapeDtypeStruct((1, 1), jnp.float32),
            jax.ShapeDtypeStruct((1, 1), jnp.float32),
        ],
        scratch_shapes=[
            pltpu.VMEM((1, NUM_K), jnp.float32),
            pltpu.VMEM((1, NUM_K), jnp.float32),
            pltpu.SMEM((1,), jnp.float32),
        ],
        compiler_params=pltpu.CompilerParams(
            dimension_semantics=("arbitrary",)),
    )(z_e, et)


def _gather_body(e_hbm, idx_hbm, out_hbm, idx_c, rows_v, sem):
    c = lax.axis_index("c")
    s = lax.axis_index("s")
    wid = s * SC_CORES + c
    base = wid * ROWS_PER_W
    for ch in range(ROWS_PER_W // GCHUNK):
        off = base + ch * GCHUNK
        pltpu.sync_copy(idx_hbm.at[pl.ds(off, GCHUNK)], idx_c)
        pltpu.async_copy(e_hbm.at[idx_c], rows_v, sem).wait()
        pltpu.sync_copy(rows_v, out_hbm.at[pl.ds(off, GCHUNK)])


def _gather_call(embeddings, indices):
    mesh = plsc.VectorSubcoreMesh(core_axis_name="c", subcore_axis_name="s")
    k = functools.partial(
        pl.kernel,
        mesh=mesh,
        out_type=jax.ShapeDtypeStruct((NUM_N, DIM), jnp.float32),
        scratch_types=[
            pltpu.VMEM((GCHUNK,), jnp.int32),
            pltpu.VMEM((GCHUNK, DIM), jnp.float32),
            pltpu.SemaphoreType.DMA,
        ],
    )(_gather_body)
    return k(embeddings, indices)


def kernel(z_e, embeddings):
    et = (embeddings + embeddings).T
    idx2, loss, perp = _dist_call(z_e, et)
    indices = idx2.reshape(NUM_N)
    z_q = _gather_call(embeddings, indices)
    return z_q, indices, loss[0, 0], perp[0, 0]
